# baseline (speedup 1.0000x reference)
import functools
from math import sqrt

import numpy as np
import jax
import jax.numpy as jnp
from jax.experimental import pallas as pl
from jax.experimental.pallas import tpu as pltpu

_D = 64
_H = 8
_E = 8
_LAYERS = 2
_FACTOR = 5
_B = 2
_L = 4096
_DFF = 256
_NCLS = 10
_U = min(_FACTOR * int(np.ceil(np.log(_L))), _L)
_UPAD = 48
_NB = 16
_RB = _L // _NB
_SCALE = 1.0 / sqrt(_E)
_PAR1 = pltpu.CompilerParams()
_PAR2 = pltpu.CompilerParams()
_NEG = -1e30


@functools.cache
def _consts():
    pos = np.arange(_L)[:, None].astype(np.float32)
    div = np.exp(np.arange(0, _D, 2).astype(np.float32) * -(np.log(10000.0) / _D))
    pe = np.zeros((_L, _D), dtype=np.float32)
    pe[:, 0::2] = np.sin(pos * div)
    pe[:, 1::2] = np.cos(pos * div)

    cnts, msks = [], []
    rows = np.arange(_L)[:, None]
    for i in range(_LAYERS):
        with jax.ensure_compile_time_eval():
            key = jax.random.key(42)
            idx = np.asarray(jax.random.randint(jax.random.fold_in(key, i),
                                                (_L, _U), 0, _L))
        cnt = np.zeros((_L, _L), dtype=np.float32)
        np.add.at(cnt, (rows, idx), 1.0)
        msk = np.where(cnt > 0, 0.0, _NEG).astype(np.float32)
        cnts.append(cnt.astype(jnp.bfloat16))
        msks.append(msk.astype(jnp.bfloat16))
    return pe, cnts, msks


def _f32dot(a, b):
    return jnp.dot(a, b, preferred_element_type=jnp.float32)


def _abt(a, b):
    return jax.lax.dot_general(a, b, (((1,), (1,)), ((), ())),
                               preferred_element_type=jnp.float32)


def _layer_norm(x, g, b, eps=1e-5):
    mu = jnp.mean(x, axis=1, keepdims=True)
    var = jnp.mean((x - mu) ** 2, axis=1, keepdims=True)
    return (x - mu) / jnp.sqrt(var + eps) * g + b


def _embed_body(x3_ref, w_ref, pe_ref, o_ref):
    o_ref[0] = _f32dot(x3_ref[0], w_ref[...]) + pe_ref[...]


def _embed(x3, w8, pe):
    return pl.pallas_call(
        _embed_body,
        grid=(_B,),
        in_specs=[
            pl.BlockSpec((1, _L, 8), lambda b: (b, 0, 0)),
            pl.BlockSpec((8, _D), lambda b: (0, 0)),
            pl.BlockSpec((_L, _D), lambda b: (0, 0)),
        ],
        out_specs=pl.BlockSpec((1, _L, _D), lambda b: (b, 0, 0)),
        out_shape=jax.ShapeDtypeStruct((_B, _L, _D), jnp.float32),
        compiler_params=_PAR1,
    )(x3, w8, pe)


def _qkv_body(h_ref, w_ref, b_ref, o_ref):
    o_ref[0] = _abt(h_ref[0], w_ref[...]) + b_ref[...]


def _qkv(h, wqkv, bqkv):
    return pl.pallas_call(
        _qkv_body,
        grid=(_B,),
        in_specs=[
            pl.BlockSpec((1, _L, _D), lambda b: (b, 0, 0)),
            pl.BlockSpec((3 * _D, _D), lambda b: (0, 0)),
            pl.BlockSpec((1, 3 * _D), lambda b: (0, 0)),
        ],
        out_specs=pl.BlockSpec((1, _L, 3 * _D), lambda b: (b, 0, 0)),
        out_shape=jax.ShapeDtypeStruct((_B, _L, 3 * _D), jnp.float32),
        compiler_params=_PAR1,
    )(h, wqkv, bqkv)


def _m_body(qm_ref, qf_ref, cnt_ref, msk_ref, m_ref):
    cnt = cnt_ref[...].astype(jnp.float32)
    msk = msk_ref[...].astype(jnp.float32)
    kfull = qf_ref[0, :, _D:2 * _D]
    ck = _f32dot(cnt, kfull)
    cols = []
    for h in range(_H):
        qb = qm_ref[0, :, h * _E:(h + 1) * _E]
        kh = kfull[:, h * _E:(h + 1) * _E]
        s = _abt(qb, kh)
        mmax = jnp.max(s + msk, axis=1, keepdims=True)
        msum = jnp.sum(qb * ck[:, h * _E:(h + 1) * _E], axis=1, keepdims=True)
        cols.append(mmax - msum * (1.0 / _L))
    m_ref[0] = jnp.concatenate(cols, axis=1)


def _m_stats(qkv, cnt, msk):
    return pl.pallas_call(
        _m_body,
        grid=(_B, _NB),
        in_specs=[
            pl.BlockSpec((1, _RB, 3 * _D), lambda b, i: (b, i, 0)),
            pl.BlockSpec((1, _L, 3 * _D), lambda b, i: (b, 0, 0)),
            pl.BlockSpec((_RB, _L), lambda b, i: (i, 0)),
            pl.BlockSpec((_RB, _L), lambda b, i: (i, 0)),
        ],
        out_specs=pl.BlockSpec((1, _RB, _H), lambda b, i: (b, i, 0)),
        out_shape=jax.ShapeDtypeStruct((_B, _L, _H), jnp.float32),
        compiler_params=_PAR2,
    )(qkv, qkv, cnt, msk)


def _eye(n):
    return (jax.lax.broadcasted_iota(jnp.int32, (n, n), 0) ==
            jax.lax.broadcasted_iota(jnp.int32, (n, n), 1)).astype(jnp.float32)


def _attn_body(m_ref, qkv_ref, o_ref):
    eye8 = _eye(_H)
    eye48 = _eye(_UPAD)
    m = _abt(eye8, m_ref[0])
    col8 = jax.lax.broadcasted_iota(jnp.int32, (_H, _L), 1)
    cols = []
    for u in range(_U):
        rmax = jnp.max(m, axis=1, keepdims=True)
        idxv = jnp.min(jnp.where(m == rmax, col8, _L), axis=1, keepdims=True)
        cols.append(idxv)
        m = jnp.where(col8 == idxv, -jnp.inf, m)
    cols += [jnp.full((_H, 1), -1, jnp.int32)] * (_UPAD - _U)
    idx_mat = jnp.concatenate(cols, axis=1).astype(jnp.float32)
    colbig = jax.lax.broadcasted_iota(jnp.int32, (_UPAD, _L), 1)
    q64 = qkv_ref[0, :, 0:_D]
    for h in range(_H):
        idxt = _abt(eye48, idx_mat[h:h + 1, :]).astype(jnp.int32)
        oh = (colbig == idxt).astype(jnp.float32)
        qr = _f32dot(oh, q64)[:, h * _E:(h + 1) * _E]
        k = qkv_ref[0, :, _D + h * _E:_D + (h + 1) * _E]
        v = qkv_ref[0, :, 2 * _D + h * _E:2 * _D + (h + 1) * _E]
        scores = _abt(qr, k) * _SCALE
        smax = jnp.max(scores, axis=1, keepdims=True)
        p = jnp.exp(scores - smax)
        attnw = p / jnp.sum(p, axis=1, keepdims=True)
        upd = _f32dot(attnw, v)
        updt = _abt(eye8, upd)
        meanvt = _abt(eye8, jnp.mean(v, axis=0, keepdims=True))
        sel = jnp.sum(oh, axis=0, keepdims=True)
        o_ref[0, h] = meanvt * (1.0 - sel) + _f32dot(updt, oh)


def _attn(m, qkv):
    return pl.pallas_call(
        _attn_body,
        grid=(_B,),
        in_specs=[
            pl.BlockSpec((1, _L, _H), lambda b: (b, 0, 0)),
            pl.BlockSpec((1, _L, 3 * _D), lambda b: (b, 0, 0)),
        ],
        out_specs=pl.BlockSpec((1, _H, _E, _L), lambda b: (b, 0, 0, 0)),
        out_shape=jax.ShapeDtypeStruct((_B, _H, _E, _L), jnp.float32),
        compiler_params=_PAR1,
    )(m, qkv)


def _ffn_body(c_ref, x_ref, wo_ref, bo_ref, g1_ref, b1_ref, w1_ref, bf1_ref,
              w2_ref, bf2_ref, g2_ref, b2_ref, o_ref):
    y = _abt(c_ref[0], wo_ref[...]) + bo_ref[...]
    xn = _layer_norm(x_ref[0] + y, g1_ref[...], b1_ref[...])
    f = jnp.maximum(_abt(xn, w1_ref[...]) + bf1_ref[...], 0.0)
    f2 = _abt(f, w2_ref[...]) + bf2_ref[...]
    o_ref[0] = _layer_norm(xn + f2, g2_ref[...], b2_ref[...])


def _ffn(ctx2, x, p):
    full = lambda shape: pl.BlockSpec(shape, lambda b: (0,) * len(shape))
    return pl.pallas_call(
        _ffn_body,
        grid=(_B,),
        in_specs=[
            pl.BlockSpec((1, _L, _D), lambda b: (b, 0, 0)),
            pl.BlockSpec((1, _L, _D), lambda b: (b, 0, 0)),
            full((_D, _D)), full((1, _D)),
            full((1, _D)), full((1, _D)),
            full((_DFF, _D)), full((1, _DFF)),
            full((_D, _DFF)), full((1, _D)),
            full((1, _D)), full((1, _D)),
        ],
        out_specs=pl.BlockSpec((1, _L, _D), lambda b: (b, 0, 0)),
        out_shape=jax.ShapeDtypeStruct((_B, _L, _D), jnp.float32),
        compiler_params=_PAR1,
    )(ctx2, x, p['Wo'], p['bo'][None], p['g1'][None], p['b1'][None],
      p['W1'], p['bf1'][None], p['W2'], p['bf2'][None],
      p['g2'][None], p['b2'][None])


def _head_body(h_ref, g_ref, b_ref, w_ref, pb_ref, o_ref):
    hn = _layer_norm(h_ref[0], g_ref[...], b_ref[...])
    logits = _abt(hn, w_ref[...]) + pb_ref[...]
    lmax = jnp.max(logits, axis=1, keepdims=True)
    p = jnp.exp(logits - lmax)
    o_ref[0] = p / jnp.sum(p, axis=1, keepdims=True)


def _head(h, ng, nb, pw, pb):
    full = lambda shape: pl.BlockSpec(shape, lambda b: (0,) * len(shape))
    return pl.pallas_call(
        _head_body,
        grid=(_B,),
        in_specs=[
            pl.BlockSpec((1, _L, _D), lambda b: (b, 0, 0)),
            full((1, _D)), full((1, _D)),
            full((_NCLS, _D)), full((1, _NCLS)),
        ],
        out_specs=pl.BlockSpec((1, _L, _NCLS), lambda b: (b, 0, 0)),
        out_shape=jax.ShapeDtypeStruct((_B, _L, _NCLS), jnp.float32),
        compiler_params=_PAR1,
    )(h, ng, nb, pw, pb)


def kernel(x_enc, params):
    pe, cnts, msks = _consts()
    x = x_enc[..., 0]
    x3 = jnp.stack([jnp.roll(x, 1, axis=1), x, jnp.roll(x, -1, axis=1)],
                   axis=-1)
    x3 = jnp.pad(x3, ((0, 0), (0, 0), (0, 5)))
    w8 = jnp.pad(jnp.transpose(params['tok_W'][:, 0, :]), ((0, 5), (0, 0)))

    h = _embed(x3, w8, jnp.asarray(pe))
    for i, p in enumerate(params['layers']):
        wqkv = jnp.concatenate([p['Wq'], p['Wk'], p['Wv']], axis=0)
        bqkv = jnp.concatenate([p['bq'], p['bk'], p['bv']])[None]
        qkv = _qkv(h, wqkv, bqkv)
        m = _m_stats(qkv, jnp.asarray(cnts[i]), jnp.asarray(msks[i]))
        ctx = _attn(m, qkv)
        ctx2 = jnp.swapaxes(ctx, 2, 3).reshape(_B, _L, _D)
        h = _ffn(ctx2, h, p)
    return _head(h, params['ng'][None], params['nb'][None],
                 params['pW'], params['pb'][None])


# speedup vs baseline: 1.0970x; 1.0970x over previous
import functools
from math import sqrt

import numpy as np
import jax
import jax.numpy as jnp
from jax.experimental import pallas as pl
from jax.experimental.pallas import tpu as pltpu

_D = 64
_H = 8
_E = 8
_LAYERS = 2
_FACTOR = 5
_B = 2
_L = 4096
_DFF = 256
_NCLS = 10
_U = min(_FACTOR * int(np.ceil(np.log(_L))), _L)
_UPAD = 48
_NB = 16
_RB = _L // _NB
_SCALE = 1.0 / sqrt(_E)
_NEG = -1e30


@functools.cache
def _consts():
    pos = np.arange(_L)[:, None].astype(np.float32)
    div = np.exp(np.arange(0, _D, 2).astype(np.float32) * -(np.log(10000.0) / _D))
    pe = np.zeros((_L, _D), dtype=np.float32)
    pe[:, 0::2] = np.sin(pos * div)
    pe[:, 1::2] = np.cos(pos * div)

    cnts, msks = [], []
    rows = np.arange(_L)[:, None]
    for i in range(_LAYERS):
        with jax.ensure_compile_time_eval():
            key = jax.random.key(42)
            idx = np.asarray(jax.random.randint(jax.random.fold_in(key, i),
                                                (_L, _U), 0, _L))
        cnt = np.zeros((_L, _L), dtype=np.float32)
        np.add.at(cnt, (rows, idx), 1.0)
        msk = np.where(cnt > 0, 0.0, _NEG).astype(np.float32)
        cnts.append(cnt.astype(jnp.bfloat16))
        msks.append(msk.astype(jnp.bfloat16))
    return pe, cnts, msks


def _f32dot(a, b):
    return jnp.dot(a, b, preferred_element_type=jnp.float32)


def _abt(a, b):
    return jax.lax.dot_general(a, b, (((1,), (1,)), ((), ())),
                               preferred_element_type=jnp.float32)


def _layer_norm(x, g, b, eps=1e-5):
    mu = jnp.mean(x, axis=1, keepdims=True)
    var = jnp.mean((x - mu) ** 2, axis=1, keepdims=True)
    return (x - mu) / jnp.sqrt(var + eps) * g + b


def _embed_body(x3_ref, w_ref, pe_ref, o_ref):
    o_ref[0] = _f32dot(x3_ref[0], w_ref[...]) + pe_ref[...]


def _embed(x3, w8, pe):
    return pl.pallas_call(
        _embed_body,
        grid=(_B,),
        in_specs=[
            pl.BlockSpec((1, _L, 8), lambda b: (b, 0, 0)),
            pl.BlockSpec((8, _D), lambda b: (0, 0)),
            pl.BlockSpec((_L, _D), lambda b: (0, 0)),
        ],
        out_specs=pl.BlockSpec((1, _L, _D), lambda b: (b, 0, 0)),
        out_shape=jax.ShapeDtypeStruct((_B, _L, _D), jnp.float32),
    )(x3, w8, pe)


def _qkv_body(h_ref, w_ref, b_ref, o_ref):
    o_ref[0] = _abt(h_ref[0], w_ref[...]) + b_ref[...]


def _qkv(h, wqkv, bqkv):
    return pl.pallas_call(
        _qkv_body,
        grid=(_B,),
        in_specs=[
            pl.BlockSpec((1, _L, _D), lambda b: (b, 0, 0)),
            pl.BlockSpec((3 * _D, _D), lambda b: (0, 0)),
            pl.BlockSpec((1, 3 * _D), lambda b: (0, 0)),
        ],
        out_specs=pl.BlockSpec((1, _L, 3 * _D), lambda b: (b, 0, 0)),
        out_shape=jax.ShapeDtypeStruct((_B, _L, 3 * _D), jnp.float32),
    )(h, wqkv, bqkv)


def _m_body(qm_ref, qf_ref, cnt_ref, msk_ref, m_ref):
    cnt = cnt_ref[...].astype(jnp.float32)
    msk = msk_ref[...].astype(jnp.float32)
    kfull = qf_ref[0, :, _D:2 * _D]
    ck = _f32dot(cnt, kfull)
    cols = []
    for h in range(_H):
        qb = qm_ref[0, :, h * _E:(h + 1) * _E]
        kh = kfull[:, h * _E:(h + 1) * _E]
        s = _abt(qb, kh)
        mmax = jnp.max(s + msk, axis=1, keepdims=True)
        msum = jnp.sum(qb * ck[:, h * _E:(h + 1) * _E], axis=1, keepdims=True)
        cols.append(mmax - msum * (1.0 / _L))
    m_ref[0] = jnp.concatenate(cols, axis=1)


def _m_stats(qkv, cnt, msk):
    return pl.pallas_call(
        _m_body,
        grid=(_B, _NB),
        in_specs=[
            pl.BlockSpec((1, _RB, 3 * _D), lambda b, i: (b, i, 0)),
            pl.BlockSpec((1, _L, 3 * _D), lambda b, i: (b, 0, 0)),
            pl.BlockSpec((_RB, _L), lambda b, i: (i, 0)),
            pl.BlockSpec((_RB, _L), lambda b, i: (i, 0)),
        ],
        out_specs=pl.BlockSpec((1, _RB, _H), lambda b, i: (b, i, 0)),
        out_shape=jax.ShapeDtypeStruct((_B, _L, _H), jnp.float32),
    )(qkv, qkv, cnt, msk)


def _topk_body(m_ref, o_ref):
    eye = (jax.lax.broadcasted_iota(jnp.int32, (_H, _H), 0) ==
           jax.lax.broadcasted_iota(jnp.int32, (_H, _H), 1)).astype(jnp.float32)
    rows = [_abt(eye, m_ref[b]) for b in range(_B)]
    m = jnp.concatenate(rows, axis=0)
    col = jax.lax.broadcasted_iota(jnp.int32, (_B * _H, _L), 1)
    o_ref[...] = jnp.zeros((_B * _H, _UPAD), jnp.int32)
    for u in range(_U):
        rmax = jnp.max(m, axis=1, keepdims=True)
        idxv = jnp.min(jnp.where(m == rmax, col, _L), axis=1, keepdims=True)
        o_ref[:, u:u + 1] = idxv
        m = jnp.where(col == idxv, -jnp.inf, m)


def _topk(m):
    return pl.pallas_call(
        _topk_body,
        out_shape=jax.ShapeDtypeStruct((_B * _H, _UPAD), jnp.int32),
    )(m)


def _attn_body(idx_ref, qkv_ref, o_ref, qr_ref):
    b = pl.program_id(0)
    qr_ref[...] = jnp.zeros((_UPAD, 3 * _D), jnp.float32)
    for h in range(_H):
        r = b * _H + h
        k = qkv_ref[0, :, _D + h * _E:_D + (h + 1) * _E]
        v = qkv_ref[0, :, 2 * _D + h * _E:2 * _D + (h + 1) * _E]
        meanv = jnp.mean(v, axis=0, keepdims=True)
        o_ref[0, h] = jnp.broadcast_to(meanv, (_L, _E))
        for u in range(_U):
            iu = idx_ref[r, u]
            qr_ref[u:u + 1, :] = qkv_ref[0, pl.ds(iu, 1), :]
        qr = qr_ref[:, h * _E:(h + 1) * _E]
        scores = _abt(qr, k) * _SCALE
        smax = jnp.max(scores, axis=1, keepdims=True)
        p = jnp.exp(scores - smax)
        attnw = p / jnp.sum(p, axis=1, keepdims=True)
        upd = _f32dot(attnw, v)
        for u in range(_U):
            iu = idx_ref[r, u]
            o_ref[0, h, pl.ds(iu, 1), :] = upd[u:u + 1, :]


def _attn(idxs, qkv):
    grid_spec = pltpu.PrefetchScalarGridSpec(
        num_scalar_prefetch=1,
        grid=(_B,),
        in_specs=[
            pl.BlockSpec((1, _L, 3 * _D), lambda b, s: (b, 0, 0)),
        ],
        out_specs=pl.BlockSpec((1, _H, _L, _E), lambda b, s: (b, 0, 0, 0)),
        scratch_shapes=[pltpu.VMEM((_UPAD, 3 * _D), jnp.float32)],
    )
    return pl.pallas_call(
        _attn_body,
        grid_spec=grid_spec,
        out_shape=jax.ShapeDtypeStruct((_B, _H, _L, _E), jnp.float32),
    )(idxs, qkv)


def _ffn_body(c_ref, x_ref, wo_ref, bo_ref, g1_ref, b1_ref, w1_ref, bf1_ref,
              w2_ref, bf2_ref, g2_ref, b2_ref, o_ref):
    y = _abt(c_ref[0], wo_ref[...]) + bo_ref[...]
    xn = _layer_norm(x_ref[0] + y, g1_ref[...], b1_ref[...])
    f = jnp.maximum(_abt(xn, w1_ref[...]) + bf1_ref[...], 0.0)
    f2 = _abt(f, w2_ref[...]) + bf2_ref[...]
    o_ref[0] = _layer_norm(xn + f2, g2_ref[...], b2_ref[...])


def _ffn(ctx2, x, p):
    full = lambda shape: pl.BlockSpec(shape, lambda b: (0,) * len(shape))
    return pl.pallas_call(
        _ffn_body,
        grid=(_B,),
        in_specs=[
            pl.BlockSpec((1, _L, _D), lambda b: (b, 0, 0)),
            pl.BlockSpec((1, _L, _D), lambda b: (b, 0, 0)),
            full((_D, _D)), full((1, _D)),
            full((1, _D)), full((1, _D)),
            full((_DFF, _D)), full((1, _DFF)),
            full((_D, _DFF)), full((1, _D)),
            full((1, _D)), full((1, _D)),
        ],
        out_specs=pl.BlockSpec((1, _L, _D), lambda b: (b, 0, 0)),
        out_shape=jax.ShapeDtypeStruct((_B, _L, _D), jnp.float32),
    )(ctx2, x, p['Wo'], p['bo'][None], p['g1'][None], p['b1'][None],
      p['W1'], p['bf1'][None], p['W2'], p['bf2'][None],
      p['g2'][None], p['b2'][None])


def _head_body(h_ref, g_ref, b_ref, w_ref, pb_ref, o_ref):
    hn = _layer_norm(h_ref[0], g_ref[...], b_ref[...])
    logits = _abt(hn, w_ref[...]) + pb_ref[...]
    lmax = jnp.max(logits, axis=1, keepdims=True)
    p = jnp.exp(logits - lmax)
    o_ref[0] = p / jnp.sum(p, axis=1, keepdims=True)


def _head(h, ng, nb, pw, pb):
    full = lambda shape: pl.BlockSpec(shape, lambda b: (0,) * len(shape))
    return pl.pallas_call(
        _head_body,
        grid=(_B,),
        in_specs=[
            pl.BlockSpec((1, _L, _D), lambda b: (b, 0, 0)),
            full((1, _D)), full((1, _D)),
            full((_NCLS, _D)), full((1, _NCLS)),
        ],
        out_specs=pl.BlockSpec((1, _L, _NCLS), lambda b: (b, 0, 0)),
        out_shape=jax.ShapeDtypeStruct((_B, _L, _NCLS), jnp.float32),
    )(h, ng, nb, pw, pb)


def kernel(x_enc, params):
    pe, cnts, msks = _consts()
    x = x_enc[..., 0]
    x3 = jnp.stack([jnp.roll(x, 1, axis=1), x, jnp.roll(x, -1, axis=1)],
                   axis=-1)
    x3 = jnp.pad(x3, ((0, 0), (0, 0), (0, 5)))
    w8 = jnp.pad(jnp.transpose(params['tok_W'][:, 0, :]), ((0, 5), (0, 0)))

    h = _embed(x3, w8, jnp.asarray(pe))
    for i, p in enumerate(params['layers']):
        wqkv = jnp.concatenate([p['Wq'], p['Wk'], p['Wv']], axis=0)
        bqkv = jnp.concatenate([p['bq'], p['bk'], p['bv']])[None]
        qkv = _qkv(h, wqkv, bqkv)
        m = _m_stats(qkv, jnp.asarray(cnts[i]), jnp.asarray(msks[i]))
        idxs = _topk(m)
        ctx = _attn(idxs, qkv)
        ctx2 = ctx.reshape(_B, _L, _D)
        h = _ffn(ctx2, h, p)
    return _head(h, params['ng'][None], params['nb'][None],
                 params['pW'], params['pb'][None])
